# Initial kernel scaffold; baseline (speedup 1.0000x reference)
import jax
import jax.numpy as jnp
from jax.experimental import pallas as pl


def kernel(hidden_states, gate_weight, e_score_correction_bias, w_gate_up, w_down, shared_gate_up, shared_down):
    raise NotImplementedError("write your pallas kernel here")


# speedup vs baseline: 2.0604x; 2.0604x over previous
import jax
import jax.numpy as jnp
from jax.experimental import pallas as pl
from jax.experimental.pallas import tpu as pltpu

T = 2048
HIDDEN = 1024
N_EXPERTS = 8
TOP_K = 2
D_FF = 512
N_GROUP = 2
PER_GROUP = N_EXPERTS // N_GROUP
RSF = 2.5

TB = 512
NTB = T // TB


def _sigmoid(x):
    return 1.0 / (1.0 + jnp.exp(-x))


def _routing_body(x_ref, gw_ref, bias_ref, combine_ref):
    x = x_ref[...]
    gw = gw_ref[...]
    logits = jax.lax.dot_general(
        x, gw, (((1,), (1,)), ((), ())), preferred_element_type=jnp.float32
    )
    scores = _sigmoid(logits)
    sfc = scores + bias_ref[...][None, :]

    def top2sum(cols):
        pairs = []
        for i in range(PER_GROUP):
            for j in range(i + 1, PER_GROUP):
                pairs.append(cols[:, i] + cols[:, j])
        return jnp.max(jnp.stack(pairs, axis=-1), axis=-1)

    gs0 = top2sum(sfc[:, :PER_GROUP])
    gs1 = top2sum(sfc[:, PER_GROUP:])
    sel0 = (gs0 >= gs1).astype(jnp.float32)
    lane = jax.lax.broadcasted_iota(jnp.int32, sfc.shape, 1)
    lanef = lane.astype(jnp.float32)
    in_g0 = (lane < PER_GROUP).astype(jnp.float32)
    selected = sel0[:, None] * in_g0 + (1.0 - sel0[:, None]) * (1.0 - in_g0)
    masked = sfc * selected - 1e9 * (1.0 - selected)

    big = float(N_EXPERTS + 1)
    m1 = jnp.max(masked, axis=1)
    idx1 = jnp.min(lanef + big * (masked != m1[:, None]).astype(jnp.float32), axis=1)
    one_sel1 = (lanef == idx1[:, None]).astype(jnp.float32)
    masked2 = masked - 2e9 * one_sel1
    m2 = jnp.max(masked2, axis=1)
    idx2 = jnp.min(lanef + big * (masked2 != m2[:, None]).astype(jnp.float32), axis=1)

    one1 = one_sel1
    one2 = (lanef == idx2[:, None]).astype(jnp.float32)
    w1 = jnp.sum(scores * one1, axis=1)
    w2 = jnp.sum(scores * one2, axis=1)
    denom = w1 + w2 + 1e-20
    w1 = w1 / denom * RSF
    w2 = w2 / denom * RSF
    combine_ref[...] = w1[:, None] * one1 + w2[:, None] * one2


def _routed_body(x_ref, comb_ref, wgu_ref, wdn_ref, out_ref):
    e = pl.program_id(1)

    @pl.when(e == 0)
    def _():
        out_ref[...] = jnp.zeros_like(out_ref)

    x = x_ref[...]
    gu = jax.lax.dot_general(
        x, wgu_ref[0], (((1,), (0,)), ((), ())), preferred_element_type=jnp.float32
    )
    g = gu[:, :D_FF]
    u = gu[:, D_FF:]
    h = g * _sigmoid(g) * u
    eo = jax.lax.dot_general(
        h, wdn_ref[0], (((1,), (0,)), ((), ())), preferred_element_type=jnp.float32
    )
    lane = jax.lax.broadcasted_iota(jnp.int32, comb_ref.shape, 1)
    cvec = jnp.sum(jnp.where(lane == e, comb_ref[...], 0.0), axis=1, keepdims=True)
    out_ref[...] += cvec * eo


def _shared_body(x_ref, sgu_ref, sdn_ref, out_ref):
    x = x_ref[...]
    gu = jax.lax.dot_general(
        x, sgu_ref[...], (((1,), (0,)), ((), ())), preferred_element_type=jnp.float32
    )
    half = gu.shape[1] // 2
    g = gu[:, :half]
    u = gu[:, half:]
    h = g * _sigmoid(g) * u
    out_ref[...] = jax.lax.dot_general(
        h, sdn_ref[...], (((1,), (0,)), ((), ())), preferred_element_type=jnp.float32
    )


def kernel(hidden_states, gate_weight, e_score_correction_bias, w_gate_up, w_down,
           shared_gate_up, shared_down):
    x = hidden_states

    combine = pl.pallas_call(
        _routing_body,
        grid=(NTB,),
        in_specs=[
            pl.BlockSpec((TB, HIDDEN), lambda t: (t, 0)),
            pl.BlockSpec((N_EXPERTS, HIDDEN), lambda t: (0, 0)),
            pl.BlockSpec((N_EXPERTS,), lambda t: (0,)),
        ],
        out_specs=pl.BlockSpec((TB, N_EXPERTS), lambda t: (t, 0)),
        out_shape=jax.ShapeDtypeStruct((T, N_EXPERTS), jnp.float32),
    )(x, gate_weight, e_score_correction_bias)

    routed = pl.pallas_call(
        _routed_body,
        grid=(NTB, N_EXPERTS),
        in_specs=[
            pl.BlockSpec((TB, HIDDEN), lambda t, e: (t, 0)),
            pl.BlockSpec((TB, N_EXPERTS), lambda t, e: (t, 0)),
            pl.BlockSpec((1, HIDDEN, 2 * D_FF), lambda t, e: (e, 0, 0)),
            pl.BlockSpec((1, D_FF, HIDDEN), lambda t, e: (e, 0, 0)),
        ],
        out_specs=pl.BlockSpec((TB, HIDDEN), lambda t, e: (t, 0)),
        out_shape=jax.ShapeDtypeStruct((T, HIDDEN), jnp.float32),
    )(x, combine, w_gate_up, w_down)

    shared = pl.pallas_call(
        _shared_body,
        grid=(NTB,),
        in_specs=[
            pl.BlockSpec((TB, HIDDEN), lambda t: (t, 0)),
            pl.BlockSpec(shared_gate_up.shape, lambda t: (0, 0)),
            pl.BlockSpec(shared_down.shape, lambda t: (0, 0)),
        ],
        out_specs=pl.BlockSpec((TB, HIDDEN), lambda t: (t, 0)),
        out_shape=jax.ShapeDtypeStruct((T, HIDDEN), jnp.float32),
    )(x, shared_gate_up, shared_down)

    return routed + shared
